# baseline (speedup 1.0000x reference)
import functools

import jax
import jax.numpy as jnp
from jax import lax
from jax.experimental import pallas as pl
from jax.experimental.pallas import tpu as pltpu
from jax.experimental.pallas import tpu_sc as plsc

_B = 16384
_P = 21
_E = 64
_N = _B * _P
_NW = 32
_PW = _N // _NW
_CH = 128
_NCH = _PW // _CH


def _sc_body(u_idx_hbm, v_idx_hbm, emb_u_hbm, emb_v_hbm, out_hbm,
             idx_u, idx_v, u_rows, v_rows, s_buf, out_v, sem):
    c = lax.axis_index("c")
    s = lax.axis_index("s")
    wid = s * 2 + c
    base = wid * _PW
    lanes = jnp.arange(16, dtype=jnp.int32)

    def chunk(i, carry):
        off = base + i * _CH
        pltpu.sync_copy(u_idx_hbm.at[pl.ds(off, _CH)], idx_u)
        pltpu.sync_copy(v_idx_hbm.at[pl.ds(off, _CH)], idx_v)
        pltpu.async_copy(emb_u_hbm.at[idx_u], u_rows, sem).wait()
        pltpu.async_copy(emb_v_hbm.at[idx_v], v_rows, sem).wait()

        def kstep(k, carry1):
            acc = (u_rows[k, pl.ds(0, 16)] * v_rows[k, pl.ds(0, 16)]
                   + u_rows[k, pl.ds(16, 16)] * v_rows[k, pl.ds(16, 16)]
                   + u_rows[k, pl.ds(32, 16)] * v_rows[k, pl.ds(32, 16)]
                   + u_rows[k, pl.ds(48, 16)] * v_rows[k, pl.ds(48, 16)])
            s_buf[pl.ds(k * 16, 16)] = acc
            return carry1

        lax.fori_loop(0, _CH, kstep, 0)

        lanes16 = lanes * 16
        for g in range(_CH // 16):
            acc = plsc.load_gather(s_buf, [lanes16 + g * 256])
            for j in range(1, 16):
                acc = acc + plsc.load_gather(s_buf, [lanes16 + (g * 256 + j)])
            out_v[pl.ds(i * _CH + g * 16, 16)] = acc
        return carry

    lax.fori_loop(0, _NCH, chunk, 0)
    pltpu.sync_copy(out_v, out_hbm.at[pl.ds(base, _PW)])


@jax.jit
def _sc_dot(u_idx, v_idx, emb_u, emb_v):
    mesh = plsc.VectorSubcoreMesh(core_axis_name="c", subcore_axis_name="s")
    f = functools.partial(
        pl.kernel,
        out_type=jax.ShapeDtypeStruct((_N,), jnp.float32),
        mesh=mesh,
        compiler_params=pltpu.CompilerParams(
            needs_layout_passes=False, use_tc_tiling_on_sc=False),
        scratch_types=[
            pltpu.VMEM((_CH,), jnp.int32),
            pltpu.VMEM((_CH,), jnp.int32),
            pltpu.VMEM((_CH, _E), jnp.float32),
            pltpu.VMEM((_CH, _E), jnp.float32),
            pltpu.VMEM((_CH * 16,), jnp.float32),
            pltpu.VMEM((_PW,), jnp.float32),
            pltpu.SemaphoreType.DMA,
        ],
    )(_sc_body)
    return f(u_idx, v_idx, emb_u, emb_v)


def kernel(x, emb_u, emb_v):
    idx = x.reshape(_N, 2)
    u_idx = idx[:, 0]
    v_idx = idx[:, 1]
    out = _sc_dot(u_idx, v_idx, emb_u, emb_v)
    return out.reshape(_B, _P)


# speedup vs baseline: 1.1737x; 1.1737x over previous
import functools

import jax
import jax.numpy as jnp
from jax import lax
from jax.experimental import pallas as pl
from jax.experimental.pallas import tpu as pltpu
from jax.experimental.pallas import tpu_sc as plsc

_B = 16384
_P = 21
_E = 64
_N = _B * _P
_NW = 32
_PW = _N // _NW
_CH = 128
_NCH = _PW // _CH


def _sc_body(u_idx_hbm, v_idx_hbm, emb_u_hbm, emb_v_hbm, out_hbm,
             uidx, vidx, u_rows0, v_rows0, u_rows1, v_rows1, s_buf, out_v,
             sem_u0, sem_v0, sem_u1, sem_v1, sem_i):
    c = lax.axis_index("c")
    s = lax.axis_index("s")
    wid = s * 2 + c
    base = wid * _PW
    lanes = jnp.arange(16, dtype=jnp.int32)
    lanes16 = lanes * 16

    cp_u = pltpu.async_copy(u_idx_hbm.at[pl.ds(base, _PW)], uidx, sem_i)
    cp_v = pltpu.async_copy(v_idx_hbm.at[pl.ds(base, _PW)], vidx, sem_i)
    cp_u.wait()
    cp_v.wait()

    bufs = ((u_rows0, v_rows0, sem_u0, sem_v0),
            (u_rows1, v_rows1, sem_u1, sem_v1))

    def start_gather(ci, slot):
        ur, vr, su, sv = bufs[slot]
        pltpu.async_copy(emb_u_hbm.at[uidx.at[pl.ds(ci * _CH, _CH)]], ur, su)
        pltpu.async_copy(emb_v_hbm.at[vidx.at[pl.ds(ci * _CH, _CH)]], vr, sv)

    def wait_gather(ci, slot):
        ur, vr, su, sv = bufs[slot]
        pltpu.make_async_copy(
            emb_u_hbm.at[uidx.at[pl.ds(ci * _CH, _CH)]], ur, su).wait()
        pltpu.make_async_copy(
            emb_v_hbm.at[vidx.at[pl.ds(ci * _CH, _CH)]], vr, sv).wait()

    def compute(ci, slot):
        ur, vr, _, _ = bufs[slot]

        @pl.loop(0, _CH, unroll=8)
        def kstep(k):
            acc = (ur[k, pl.ds(0, 16)] * vr[k, pl.ds(0, 16)]
                   + ur[k, pl.ds(16, 16)] * vr[k, pl.ds(16, 16)]
                   + ur[k, pl.ds(32, 16)] * vr[k, pl.ds(32, 16)]
                   + ur[k, pl.ds(48, 16)] * vr[k, pl.ds(48, 16)])
            s_buf[pl.ds(k * 16, 16)] = acc

        for g in range(_CH // 16):
            acc = plsc.load_gather(s_buf, [lanes16 + g * 256])
            for j in range(1, 16):
                acc = acc + plsc.load_gather(s_buf, [lanes16 + (g * 256 + j)])
            out_v[pl.ds(ci * _CH + g * 16, 16)] = acc

    start_gather(0, 0)

    @pl.loop(0, _NCH // 2)
    def group(g):
        a = 2 * g
        b = a + 1
        wait_gather(a, 0)
        start_gather(b, 1)
        compute(a, 0)
        wait_gather(b, 1)
        start_gather(jnp.minimum(a + 2, _NCH - 1), 0)
        compute(b, 1)

    wait_gather(_NCH - 1, 0)
    pltpu.sync_copy(out_v, out_hbm.at[pl.ds(base, _PW)])


@jax.jit
def _sc_dot(u_idx, v_idx, emb_u, emb_v):
    mesh = plsc.VectorSubcoreMesh(core_axis_name="c", subcore_axis_name="s")
    f = functools.partial(
        pl.kernel,
        out_type=jax.ShapeDtypeStruct((_N,), jnp.float32),
        mesh=mesh,
        compiler_params=pltpu.CompilerParams(
            needs_layout_passes=False, use_tc_tiling_on_sc=False),
        scratch_types=[
            pltpu.VMEM((_PW,), jnp.int32),
            pltpu.VMEM((_PW,), jnp.int32),
            pltpu.VMEM((_CH, _E), jnp.float32),
            pltpu.VMEM((_CH, _E), jnp.float32),
            pltpu.VMEM((_CH, _E), jnp.float32),
            pltpu.VMEM((_CH, _E), jnp.float32),
            pltpu.VMEM((_CH * 16,), jnp.float32),
            pltpu.VMEM((_PW,), jnp.float32),
            pltpu.SemaphoreType.DMA,
            pltpu.SemaphoreType.DMA,
            pltpu.SemaphoreType.DMA,
            pltpu.SemaphoreType.DMA,
            pltpu.SemaphoreType.DMA,
        ],
    )(_sc_body)
    return f(u_idx, v_idx, emb_u, emb_v)


def kernel(x, emb_u, emb_v):
    idx = x.reshape(_N, 2)
    u_idx = idx[:, 0]
    v_idx = idx[:, 1]
    out = _sc_dot(u_idx, v_idx, emb_u, emb_v)
    return out.reshape(_B, _P)
